# baseline (speedup 1.0000x reference)
import functools

import jax
import jax.numpy as jnp
from jax.experimental import pallas as pl
from jax.experimental.pallas import tpu as pltpu

N_BINS = 8
SHARP = 25.0 * N_BINS
ROWS_PER_STEP = 2048


def _fused_kernel(x_ref, w_ref, b_ref, z_ref, hist_ref, acc_ref, *, nsteps, n_samples):
    i = pl.program_id(0)

    @pl.when(i == 0)
    def _init():
        acc_ref[...] = jnp.zeros_like(acc_ref)

    p = jnp.dot(x_ref[...], w_ref[...], preferred_element_type=jnp.float32)
    p = p + b_ref[0, :][None, :]
    t0 = jnp.tanh(0.5 * p)
    z_ref[...] = 0.5 * t0 + 0.5
    zz = (0.25 * SHARP) * t0 + (0.25 * SHARP)

    r = zz.shape[0]
    for k in range(N_BINS + 1):
        t = jnp.tanh(zz - (0.5 * SHARP / N_BINS) * k)
        acc_ref[k] += t.reshape(r // 8, 8, 128).sum(axis=0)

    @pl.when(i == nsteps - 1)
    def _finalize():
        a = acc_ref[...].sum(axis=1)
        a = a[:, :64] + a[:, 64:]
        h = (a[0:N_BINS, :] - a[1 : N_BINS + 1, :]) * (0.5 / n_samples)
        denom = jnp.maximum(h.sum(axis=0, keepdims=True), 1e-6)
        hist_ref[...] = (h / denom).T


def kernel(X, W, b, attention):
    del attention
    n, in_dim = X.shape
    d = W.shape[1]
    xr = X.reshape(n // 2, 2 * in_dim)
    zpad = jnp.zeros_like(W)
    w2 = jnp.concatenate(
        [jnp.concatenate([W, zpad], axis=1), jnp.concatenate([zpad, W], axis=1)],
        axis=0,
    )
    b2 = jnp.broadcast_to(jnp.concatenate([b, b])[None, :], (8, 2 * d))

    nsteps = (n // 2) // ROWS_PER_STEP
    z2, hist = pl.pallas_call(
        functools.partial(_fused_kernel, nsteps=nsteps, n_samples=n),
        grid=(nsteps,),
        in_specs=[
            pl.BlockSpec((ROWS_PER_STEP, 2 * in_dim), lambda i: (i, 0)),
            pl.BlockSpec((2 * in_dim, 2 * d), lambda i: (0, 0)),
            pl.BlockSpec((8, 2 * d), lambda i: (0, 0)),
        ],
        out_specs=[
            pl.BlockSpec((ROWS_PER_STEP, 2 * d), lambda i: (i, 0)),
            pl.BlockSpec((d, N_BINS), lambda i: (0, 0)),
        ],
        out_shape=[
            jax.ShapeDtypeStruct((n // 2, 2 * d), jnp.float32),
            jax.ShapeDtypeStruct((d, N_BINS), jnp.float32),
        ],
        scratch_shapes=[pltpu.VMEM((N_BINS + 1, 8, 128), jnp.float32)],
    )(xr, w2, b2)

    return (hist.reshape(-1), z2.reshape(n, d))
